# baseline (speedup 1.0000x reference)
import functools

import jax
import jax.numpy as jnp
from jax import lax
from jax.experimental import pallas as pl
from jax.experimental.pallas import tpu as pltpu
from jax.experimental.pallas import tpu_sc as plsc

_D = 64
_SEQ = 200
_ROWS = 7
_NW = 14
_CHUNK = 104
_LANES = 16
_NCH = _D // _LANES


def _emb_body(ids_hbm, wexp_hbm, table_hbm, out_hbm,
              idx_v, rows_v, partial_v, shared, gath_v, wexp_v, sem):
    sid = lax.axis_index("s")

    @pl.when(sid < _NW)
    def _():
        limit = jnp.where(sid % 2 == 0, 104, 96)

        pltpu.sync_copy(ids_hbm.at[sid], idx_v)
        pltpu.async_copy(table_hbm.at[idx_v], rows_v, sem).wait()

        def body(t, acc):
            return tuple(acc[c] + rows_v[t, pl.ds(c * _LANES, _LANES)]
                         for c in range(_NCH))

        acc = lax.fori_loop(
            0, limit, body,
            tuple(jnp.zeros((_LANES,), jnp.float32) for _ in range(_NCH)))
        for c in range(_NCH):
            partial_v[pl.ds(c * _LANES, _LANES)] = acc[c]
        pltpu.sync_copy(partial_v, shared.at[sid])

    plsc.subcore_barrier()

    @pl.when(sid == 0)
    def _():
        pltpu.sync_copy(shared, gath_v)
        pltpu.sync_copy(wexp_hbm, wexp_v)

        def rbody(i, acc):
            return tuple(acc[c] + wexp_v[i, pl.ds(c * _LANES, _LANES)]
                         * gath_v[i, pl.ds(c * _LANES, _LANES)]
                         for c in range(_NCH))

        racc = lax.fori_loop(
            0, _NW, rbody,
            tuple(jnp.zeros((_LANES,), jnp.float32) for _ in range(_NCH)))
        for c in range(_NCH):
            partial_v[pl.ds(c * _LANES, _LANES)] = racc[c]
        pltpu.sync_copy(partial_v, out_hbm.at[0])


_emb_kernel = functools.partial(
    pl.kernel,
    out_type=jax.ShapeDtypeStruct((1, _D), jnp.float32),
    mesh=plsc.VectorSubcoreMesh(
        core_axis_name="c", subcore_axis_name="s", num_cores=1),
    compiler_params=pltpu.CompilerParams(use_tc_tiling_on_sc=False),
    scratch_types=[
        pltpu.VMEM((_CHUNK,), jnp.int32),
        pltpu.VMEM((_CHUNK, _D), jnp.float32),
        pltpu.VMEM((_D,), jnp.float32),
        pltpu.VMEM_SHARED((_NW, _D), jnp.float32),
        pltpu.VMEM((_NW, _D), jnp.float32),
        pltpu.VMEM((_NW, _D), jnp.float32),
        pltpu.SemaphoreType.DMA,
    ],
)(_emb_body)


def kernel(inputs, table):
    ratio = inputs[:, 0]
    ids = inputs[:, 1:].astype(jnp.int32)
    ids_p = jnp.concatenate(
        [ids, jnp.zeros((_ROWS, 2 * _CHUNK - _SEQ), jnp.int32)],
        axis=1).reshape(_NW, _CHUNK)
    wexp = jnp.broadcast_to(
        jnp.repeat(ratio * (1.0 / _SEQ), 2)[:, None], (_NW, _D))
    return _emb_kernel(ids_p, wexp, table)


# speedup vs baseline: 4.6294x; 4.6294x over previous
import functools

import jax
import jax.numpy as jnp
from jax import lax
from jax.experimental import pallas as pl
from jax.experimental.pallas import tpu as pltpu
from jax.experimental.pallas import tpu_sc as plsc

_D = 64
_SEQ = 200
_ROWS = 7
_NW = 14
_CHUNK = 112
_LANES = 16
_NCH = _D // _LANES


def _emb_body(ids_hbm, wexp_hbm, tableT_hbm, out_hbm,
              idx_v, slab_v, partial_v, shared, gath_v, wexp_v, sem):
    sid = lax.axis_index("s")

    @pl.when(sid < _NW)
    def _():
        limit = jnp.where(sid % 2 == 0, 112, 88)
        nchunks = jnp.where(sid % 2 == 0, 7, 6)

        pltpu.sync_copy(ids_hbm.at[sid], idx_v)
        iota = lax.iota(jnp.int32, _LANES)

        def chunk_body(ck, acc):
            base = ck * _LANES
            chunk = idx_v[pl.ds(base, _LANES)]
            acc = list(acc)
            for j in range(_LANES):
                idj = jnp.sum(jnp.where(iota == j, chunk, 0))
                tile_col = pl.multiple_of((idj // 128) * 128, 128)
                lane = idj % 128
                pltpu.async_copy(
                    tableT_hbm.at[:, pl.ds(tile_col, 128)], slab_v, sem
                ).wait()
                valid = base + j < limit
                lanes = jnp.full((_LANES,), lane, jnp.int32)
                for c in range(_NCH):
                    col = plsc.load_gather(
                        slab_v, [iota + c * _LANES, lanes])
                    acc[c] = acc[c] + jnp.where(valid, col, 0.0)
            return tuple(acc)

        acc = lax.fori_loop(
            0, nchunks, chunk_body,
            tuple(jnp.zeros((_LANES,), jnp.float32) for _ in range(_NCH)))
        for c in range(_NCH):
            partial_v[pl.ds(c * _LANES, _LANES)] = acc[c]
        pltpu.sync_copy(partial_v, shared.at[sid])

    plsc.subcore_barrier()

    @pl.when(sid == 0)
    def _():
        pltpu.sync_copy(shared.at[pl.ds(0, _NW)], gath_v)
        pltpu.sync_copy(wexp_hbm, wexp_v)

        def rbody(i, acc):
            return tuple(acc[c] + wexp_v[i, pl.ds(c * _LANES, _LANES)]
                         * gath_v[i, pl.ds(c * _LANES, _LANES)]
                         for c in range(_NCH))

        racc = lax.fori_loop(
            0, _NW, rbody,
            tuple(jnp.zeros((_LANES,), jnp.float32) for _ in range(_NCH)))
        for c in range(_NCH):
            partial_v[pl.ds(c * _LANES, _LANES)] = racc[c]
        pltpu.sync_copy(partial_v, out_hbm.at[0])


_emb_kernel = functools.partial(
    pl.kernel,
    out_type=jax.ShapeDtypeStruct((1, _D), jnp.float32),
    mesh=plsc.VectorSubcoreMesh(
        core_axis_name="c", subcore_axis_name="s", num_cores=1),
    compiler_params=pltpu.CompilerParams(needs_layout_passes=False),
    scratch_types=[
        pltpu.VMEM((_CHUNK,), jnp.int32),
        pltpu.VMEM((_D, 128), jnp.float32),
        pltpu.VMEM((_D,), jnp.float32),
        pltpu.VMEM_SHARED((2 * _NW, _D), jnp.float32),
        pltpu.VMEM((_NW, _D), jnp.float32),
        pltpu.VMEM((_NW, _D), jnp.float32),
        pltpu.SemaphoreType.DMA,
    ],
)(_emb_body)


def kernel(inputs, table):
    ratio = inputs[:, 0]
    ids = inputs[:, 1:].astype(jnp.int32)
    ids_p = jnp.concatenate(
        [ids, jnp.zeros((_ROWS, 2 * _CHUNK - _SEQ), jnp.int32)],
        axis=1).reshape(_NW, _CHUNK)
    wexp = jnp.broadcast_to(
        jnp.repeat(ratio * (1.0 / _SEQ), 2)[:, None], (_NW, _D))
    return _emb_kernel(ids_p, wexp, table.T)


# speedup vs baseline: 9.2543x; 1.9990x over previous
import functools

import jax
import jax.numpy as jnp
from jax import lax
from jax.experimental import pallas as pl
from jax.experimental.pallas import tpu as pltpu
from jax.experimental.pallas import tpu_sc as plsc

_D = 64
_SEQ = 200
_ROWS = 7
_NW = 14
_CHUNK = 112
_LANES = 16
_NCH = _D // _LANES
_K = 4


def _emb_body(ids_hbm, wexp_hbm, tableT_hbm, out_hbm,
              idx_v, slab0_v, slab1_v, partial_v, shared, gath_v, wexp_v,
              sem0, sem1):
    sid = lax.axis_index("s")

    @pl.when(sid < _NW)
    def _():
        nb = jnp.where(sid % 2 == 0, 28, 22)
        pltpu.sync_copy(ids_hbm.at[sid], idx_v)
        iota = lax.iota(jnp.int32, _LANES)

        def batch_ids(b):
            chunk = idx_v[pl.ds(((_K * b) // _LANES) * _LANES, _LANES)]
            jo = (_K * b) % _LANES
            return [jnp.sum(jnp.where(iota == jo + k, chunk, 0))
                    for k in range(_K)]

        def issue(b, slab, sem):
            for k, idj in enumerate(batch_ids(b)):
                tile_col = pl.multiple_of((idj // 128) * 128, 128)
                pltpu.async_copy(
                    tableT_hbm.at[:, pl.ds(tile_col, 128)], slab.at[k], sem)

        def drain(slab, sem):
            for k in range(_K):
                pltpu.make_async_copy(
                    tableT_hbm.at[:, pl.ds(0, 128)], slab.at[k], sem).wait()

        def process(b, slab, acc):
            acc = list(acc)
            for k, idj in enumerate(batch_ids(b)):
                lanes = jnp.full((_LANES,), idj % 128, jnp.int32)
                for c in range(_NCH):
                    col = plsc.load_gather(
                        slab.at[k], [iota + c * _LANES, lanes])
                    acc[c] = acc[c] + col
            return tuple(acc)

        issue(0, slab0_v, sem0)

        def body(b2, acc):
            b = 2 * b2
            issue(b + 1, slab1_v, sem1)
            drain(slab0_v, sem0)
            acc = process(b, slab0_v, acc)

            @pl.when(b + 2 < nb)
            def _():
                issue(b + 2, slab0_v, sem0)

            drain(slab1_v, sem1)
            return process(b + 1, slab1_v, acc)

        acc = lax.fori_loop(
            0, nb // 2, body,
            tuple(jnp.zeros((_LANES,), jnp.float32) for _ in range(_NCH)))
        for c in range(_NCH):
            partial_v[pl.ds(c * _LANES, _LANES)] = acc[c]
        pltpu.sync_copy(partial_v, shared.at[sid])

    plsc.subcore_barrier()

    @pl.when(sid == 0)
    def _():
        pltpu.sync_copy(shared.at[pl.ds(0, _NW)], gath_v)
        pltpu.sync_copy(wexp_hbm, wexp_v)

        def rbody(i, acc):
            return tuple(acc[c] + wexp_v[i, pl.ds(c * _LANES, _LANES)]
                         * gath_v[i, pl.ds(c * _LANES, _LANES)]
                         for c in range(_NCH))

        racc = lax.fori_loop(
            0, _NW, rbody,
            tuple(jnp.zeros((_LANES,), jnp.float32) for _ in range(_NCH)))
        for c in range(_NCH):
            partial_v[pl.ds(c * _LANES, _LANES)] = racc[c]
        pltpu.sync_copy(partial_v, out_hbm.at[0])


_emb_kernel = functools.partial(
    pl.kernel,
    out_type=jax.ShapeDtypeStruct((1, _D), jnp.float32),
    mesh=plsc.VectorSubcoreMesh(
        core_axis_name="c", subcore_axis_name="s", num_cores=1),
    compiler_params=pltpu.CompilerParams(needs_layout_passes=False),
    scratch_types=[
        pltpu.VMEM((_CHUNK,), jnp.int32),
        pltpu.VMEM((_K, _D, 128), jnp.float32),
        pltpu.VMEM((_K, _D, 128), jnp.float32),
        pltpu.VMEM((_D,), jnp.float32),
        pltpu.VMEM_SHARED((2 * _NW, _D), jnp.float32),
        pltpu.VMEM((_NW, _D), jnp.float32),
        pltpu.VMEM((_NW, _D), jnp.float32),
        pltpu.SemaphoreType.DMA,
        pltpu.SemaphoreType.DMA,
    ],
)(_emb_body)


def kernel(inputs, table):
    ratio = inputs[:, 0]
    ids = inputs[:, 1:].astype(jnp.int32)
    ids_p = jnp.concatenate(
        [ids, jnp.zeros((_ROWS, 2 * _CHUNK - _SEQ), jnp.int32)],
        axis=1).reshape(_NW, _CHUNK)
    wexp = jnp.broadcast_to(
        jnp.repeat(ratio * (1.0 / _SEQ), 2)[:, None], (_NW, _D))
    return _emb_kernel(ids_p, wexp, table.T)


# speedup vs baseline: 12.5127x; 1.3521x over previous
import functools

import jax
import jax.numpy as jnp
from jax import lax
from jax.experimental import pallas as pl
from jax.experimental.pallas import tpu as pltpu
from jax.experimental.pallas import tpu_sc as plsc

_D = 64
_DH = 32
_SEQ = 200
_ROWS = 7
_NW = 14
_CHUNK = 112
_LANES = 16
_NCH = _DH // _LANES
_K = 4


def _emb_body(ids_hbm, tableT_hbm, out_hbm,
              idx_v, slab0_v, slab1_v, partial_v, sem0, sem1):
    sid = lax.axis_index("s")
    cid = lax.axis_index("c")
    doff = pl.multiple_of(cid * _DH, _DH)

    @pl.when(sid < _NW)
    def _():
        nb = jnp.where(sid % 2 == 0, 28, 22)
        pltpu.sync_copy(ids_hbm.at[sid], idx_v)
        iota = lax.iota(jnp.int32, _LANES)

        def batch_ids(b):
            chunk = idx_v[pl.ds(((_K * b) // _LANES) * _LANES, _LANES)]
            jo = (_K * b) % _LANES
            return [jnp.sum(jnp.where(iota == jo + k, chunk, 0))
                    for k in range(_K)]

        def issue(b, slab, sem):
            for k, idj in enumerate(batch_ids(b)):
                tile_col = pl.multiple_of((idj // 128) * 128, 128)
                pltpu.async_copy(
                    tableT_hbm.at[pl.ds(doff, _DH), pl.ds(tile_col, 128)],
                    slab.at[k], sem)

        def drain(slab, sem):
            for k in range(_K):
                pltpu.make_async_copy(
                    tableT_hbm.at[pl.ds(0, _DH), pl.ds(0, 128)],
                    slab.at[k], sem).wait()

        def process(b, slab, acc):
            acc = list(acc)
            for k, idj in enumerate(batch_ids(b)):
                lanes = jnp.full((_LANES,), idj % 128, jnp.int32)
                for c in range(_NCH):
                    col = plsc.load_gather(
                        slab.at[k], [iota + c * _LANES, lanes])
                    acc[c] = acc[c] + col
            return tuple(acc)

        issue(0, slab0_v, sem0)

        def body(b2, acc):
            b = 2 * b2
            issue(b + 1, slab1_v, sem1)
            drain(slab0_v, sem0)
            acc = process(b, slab0_v, acc)

            @pl.when(b + 2 < nb)
            def _():
                issue(b + 2, slab0_v, sem0)

            drain(slab1_v, sem1)
            return process(b + 1, slab1_v, acc)

        acc = lax.fori_loop(
            0, nb // 2, body,
            tuple(jnp.zeros((_LANES,), jnp.float32) for _ in range(_NCH)))
        for c in range(_NCH):
            partial_v[pl.ds(c * _LANES, _LANES)] = acc[c]
        pltpu.sync_copy(partial_v, out_hbm.at[cid * 16 + sid])

    @pl.when(sid >= _NW)
    def _():
        for c in range(_NCH):
            partial_v[pl.ds(c * _LANES, _LANES)] = jnp.zeros(
                (_LANES,), jnp.float32)
        pltpu.sync_copy(partial_v, out_hbm.at[cid * 16 + sid])


_emb_kernel = functools.partial(
    pl.kernel,
    out_type=jax.ShapeDtypeStruct((32, _DH), jnp.float32),
    mesh=plsc.VectorSubcoreMesh(
        core_axis_name="c", subcore_axis_name="s"),
    compiler_params=pltpu.CompilerParams(needs_layout_passes=False),
    scratch_types=[
        pltpu.VMEM((_CHUNK,), jnp.int32),
        pltpu.VMEM((_K, _DH, 128), jnp.float32),
        pltpu.VMEM((_K, _DH, 128), jnp.float32),
        pltpu.VMEM((_DH,), jnp.float32),
        pltpu.SemaphoreType.DMA,
        pltpu.SemaphoreType.DMA,
    ],
)(_emb_body)


def _reduce_body(p_ref, w_ref, o_ref):
    wp = p_ref[...] * w_ref[...]
    lo = jnp.sum(wp[0:16, :], axis=0)
    hi = jnp.sum(wp[16:32, :], axis=0)
    o_ref[...] = jnp.concatenate([lo, hi])[None, :]


_reduce_kernel = pl.pallas_call(
    _reduce_body,
    out_shape=jax.ShapeDtypeStruct((1, _D), jnp.float32),
)


def kernel(inputs, table):
    ratio = inputs[:, 0]
    ids = inputs[:, 1:].astype(jnp.int32)
    ids_p = jnp.concatenate(
        [ids, jnp.zeros((_ROWS, 2 * _CHUNK - _SEQ), jnp.int32)],
        axis=1).reshape(_NW, _CHUNK)
    partials = _emb_kernel(ids_p, table.T)
    w14 = jnp.repeat(ratio * (1.0 / _SEQ), 2)
    w16 = jnp.concatenate([w14, jnp.zeros((2,), jnp.float32)])
    w32 = jnp.concatenate([w16, w16])[:, None]
    return _reduce_kernel(partials, w32)
